# baseline (speedup 1.0000x reference)
import jax
import jax.numpy as jnp
from jax.experimental import pallas as pl


def _norm_kernel(x_ref, out_ref):
    x = x_ref[...]
    s = jnp.sum(x * x, axis=2)
    out_ref[...] = jnp.sqrt(s)[:, None, :]


def _norms(x):
    B, L, C = x.shape
    LBLK = 2048
    NL = L // LBLK
    out = pl.pallas_call(
        _norm_kernel,
        grid=(B, NL),
        in_specs=[pl.BlockSpec((1, LBLK, C), lambda b, l: (b, l, 0))],
        out_specs=pl.BlockSpec((1, 1, LBLK), lambda b, l: (b * NL + l, 0, 0)),
        out_shape=jax.ShapeDtypeStruct((B * NL, 1, LBLK), jnp.float32),
    )(x)
    return out.reshape(B, L)


def kernel(x):
    B, L, C = x.shape
    K = L // 2
    xn = _norms(x)
    _, top_idx = jax.lax.top_k(xn, K)
    idx = jnp.sort(top_idx, axis=1)
    xs = jnp.take_along_axis(x, idx[:, :, None], axis=1)
    return (xs, idx)


# speedup vs baseline: 1.3580x; 1.3580x over previous
import jax
import jax.numpy as jnp
from jax import lax
from jax.experimental import pallas as pl
from jax.experimental.pallas import tpu as pltpu
from jax.experimental.pallas import tpu_sc as plsc

B, L, C = 4, 8192, 768
K = L // 2
LBLK = 2048
NL = L // LBLK
STAGE_ROW = K + 128


def _tc_body(x_ref, keys_ref, tval_ref, rval_ref):
    l = pl.program_id(1)
    x = x_ref[...]
    n = jnp.sqrt(jnp.sum(x * x, axis=2))
    keys_ref[:, :, pl.ds(l * LBLK, LBLK)] = n[:, None, :]

    @pl.when(l == NL - 1)
    def _():
        kb = lax.bitcast_convert_type(keys_ref[...], jnp.int32)

        def body(i, t):
            cand = t | (jnp.int32(1) << (30 - i))
            cnt = jnp.sum((kb >= cand).astype(jnp.int32))
            return jnp.where(cnt >= K, cand, t)

        t_bits = lax.fori_loop(0, 31, body, jnp.int32(0))
        cgt = jnp.sum((kb > t_bits).astype(jnp.int32))
        tval_ref[...] = jnp.full(
            (1, 1, 128), lax.bitcast_convert_type(t_bits, jnp.float32))
        rval_ref[...] = jnp.full((1, 1, 128), K - cgt, jnp.int32)


def _tc_stage(x):
    keys, tval, rval = pl.pallas_call(
        _tc_body,
        grid=(B, NL),
        in_specs=[pl.BlockSpec((1, LBLK, C), lambda b, l: (b, l, 0))],
        out_specs=[
            pl.BlockSpec((1, 1, L), lambda b, l: (b, 0, 0)),
            pl.BlockSpec((1, 1, 128), lambda b, l: (b, 0, 0)),
            pl.BlockSpec((1, 1, 128), lambda b, l: (b, 0, 0)),
        ],
        out_shape=[
            jax.ShapeDtypeStruct((B, 1, L), jnp.float32),
            jax.ShapeDtypeStruct((B, 1, 128), jnp.float32),
            jax.ShapeDtypeStruct((B, 1, 128), jnp.int32),
        ],
    )(x)
    return keys.reshape(B, L), tval.reshape(B, 128), rval.reshape(B, 128)


def _sc_body(x_ref, keys_ref, tval_ref, rval_ref,
             xs_ref, idx_ref,
             keys_v, tv, rv, posb, valb, idxmy, rows_a, rows_b, stage_sh, gsem):
    c = lax.axis_index("c")
    s = lax.axis_index("s")
    bloc = s // 8
    shard = s % 8
    b = 2 * c + bloc
    start = shard * 1024

    pltpu.sync_copy(keys_ref.at[b], keys_v)
    pltpu.sync_copy(tval_ref.at[b], tv)
    pltpu.sync_copy(rval_ref.at[b], rv)
    tvec = tv[pl.ds(0, 16)]
    rvec = rv[pl.ds(0, 16)]
    lane = lax.iota(jnp.int32, 16)

    def pbody(j, carry):
        gt, eq = carry
        kv = keys_v[pl.ds(j * 16, 16)]
        gt = gt + jnp.sum((kv > tvec).astype(jnp.int32))
        eq = eq + jnp.sum((kv == tvec).astype(jnp.int32))
        return gt, eq

    zeros = jnp.zeros((16,), jnp.int32)
    gt0, eq0 = lax.fori_loop(0, shard * 64, pbody, (zeros, zeros))

    off = gt0 + jnp.minimum(eq0, rvec)
    eqs = eq0
    for j in range(64):
        kv = keys_v[pl.ds(start + j * 16, 16)]
        gtm = kv > tvec
        eqm = kv == tvec
        eqi = eqm.astype(jnp.int32)
        eq_excl = plsc.cumsum(eqi) - eqi
        sel = gtm | (eqm & ((eqs + eq_excl) < rvec))
        seli = sel.astype(jnp.int32)
        rank = plsc.cumsum(seli) - seli
        dump = bloc * STAGE_ROW + K + ((s * 16 + lane) % 128)
        pos = jnp.where(sel, bloc * STAGE_ROW + off + rank, dump)
        posb[j // 8, pl.ds((j % 8) * 16, 16)] = pos
        valb[j // 8, pl.ds((j % 8) * 16, 16)] = start + j * 16 + lane
        off = off + jnp.sum(seli)
        eqs = eqs + jnp.sum(eqi)

    for t in range(8):
        pltpu.sync_copy(valb.at[t], stage_sh.at[posb.at[t]])

    plsc.subcore_barrier()

    kbase = shard * 512
    pltpu.sync_copy(stage_sh.at[pl.ds(bloc * STAGE_ROW + kbase, 512)], idxmy)
    pltpu.sync_copy(idxmy, idx_ref.at[b, pl.ds(kbase, 512)])

    bufs = (rows_a, rows_b)
    desc = pltpu.async_copy(
        x_ref.at[b].at[idxmy.at[pl.ds(0, 64)]], bufs[0], gsem)
    for t in range(8):
        desc.wait()
        cur = bufs[t % 2]
        if t < 7:
            desc = pltpu.async_copy(
                x_ref.at[b].at[idxmy.at[pl.ds((t + 1) * 64, 64)]],
                bufs[(t + 1) % 2], gsem)
        pltpu.sync_copy(cur, xs_ref.at[b, pl.ds(kbase + t * 64, 64), :])


def _sc_stage(x, keys, tval, rval):
    mesh = plsc.VectorSubcoreMesh(core_axis_name="c", subcore_axis_name="s")
    out_type = (
        jax.ShapeDtypeStruct((B, K, C), jnp.float32),
        jax.ShapeDtypeStruct((B, K), jnp.int32),
    )
    xs, idx = pl.kernel(
        _sc_body,
        out_type,
        mesh=mesh,
        compiler_params=pltpu.CompilerParams(needs_layout_passes=False),
        scratch_types=[
            pltpu.VMEM((L,), jnp.float32),
            pltpu.VMEM((128,), jnp.float32),
            pltpu.VMEM((128,), jnp.int32),
            pltpu.VMEM((8, 128), jnp.int32),
            pltpu.VMEM((8, 128), jnp.int32),
            pltpu.VMEM((512,), jnp.int32),
            pltpu.VMEM((64, C), jnp.float32),
            pltpu.VMEM((64, C), jnp.float32),
            pltpu.VMEM_SHARED((2 * STAGE_ROW,), jnp.int32),
            pltpu.SemaphoreType.DMA,
        ],
    )(x, keys, tval, rval)
    return xs, idx


def kernel(x):
    keys, tval, rval = _tc_stage(x)
    xs, idx = _sc_stage(x, keys, tval, rval)
    return (xs, idx)


# speedup vs baseline: 1.3711x; 1.0096x over previous
import jax
import jax.numpy as jnp
from jax import lax
from jax.experimental import pallas as pl
from jax.experimental.pallas import tpu as pltpu
from jax.experimental.pallas import tpu_sc as plsc

B, L, C = 4, 8192, 768
K = L // 2
LBLK = 8192
NL = L // LBLK
STAGE_ROW = K + 128


def _tc_body(x_ref, keys_ref, tval_ref, rval_ref):
    l = pl.program_id(1)
    x = x_ref[...]
    n = jnp.sqrt(jnp.sum(x * x, axis=2))
    keys_ref[:, :, pl.ds(l * LBLK, LBLK)] = n[:, None, :]

    @pl.when(l == NL - 1)
    def _():
        kb = lax.bitcast_convert_type(keys_ref[...], jnp.int32)

        def body(i, t):
            cand = t | (jnp.int32(1) << (30 - i))
            cnt = jnp.sum((kb >= cand).astype(jnp.int32))
            return jnp.where(cnt >= K, cand, t)

        t_bits = lax.fori_loop(0, 31, body, jnp.int32(0))
        cgt = jnp.sum((kb > t_bits).astype(jnp.int32))
        tval_ref[...] = jnp.full(
            (1, 1, 128), lax.bitcast_convert_type(t_bits, jnp.float32))
        rval_ref[...] = jnp.full((1, 1, 128), K - cgt, jnp.int32)


def _tc_stage(x):
    keys, tval, rval = pl.pallas_call(
        _tc_body,
        grid=(B, NL),
        in_specs=[pl.BlockSpec((1, LBLK, C), lambda b, l: (b, l, 0))],
        out_specs=[
            pl.BlockSpec((1, 1, L), lambda b, l: (b, 0, 0)),
            pl.BlockSpec((1, 1, 128), lambda b, l: (b, 0, 0)),
            pl.BlockSpec((1, 1, 128), lambda b, l: (b, 0, 0)),
        ],
        out_shape=[
            jax.ShapeDtypeStruct((B, 1, L), jnp.float32),
            jax.ShapeDtypeStruct((B, 1, 128), jnp.float32),
            jax.ShapeDtypeStruct((B, 1, 128), jnp.int32),
        ],
    )(x)
    return keys.reshape(B, L), tval.reshape(B, 128), rval.reshape(B, 128)


def _sc_body(x_ref, keys_ref, tval_ref, rval_ref,
             xs_ref, idx_ref,
             keys_v, tv, rv, posb, valb, idxmy, rows_a, rows_b, stage_sh, gsem):
    c = lax.axis_index("c")
    s = lax.axis_index("s")
    bloc = s // 8
    shard = s % 8
    b = 2 * c + bloc
    start = shard * 1024

    pltpu.sync_copy(keys_ref.at[b], keys_v)
    pltpu.sync_copy(tval_ref.at[b], tv)
    pltpu.sync_copy(rval_ref.at[b], rv)
    tvec = tv[pl.ds(0, 16)]
    rvec = rv[pl.ds(0, 16)]
    lane = lax.iota(jnp.int32, 16)

    def pbody(j, carry):
        gt, eq = carry
        kv = keys_v[pl.ds(j * 16, 16)]
        gt = gt + jnp.sum((kv > tvec).astype(jnp.int32))
        eq = eq + jnp.sum((kv == tvec).astype(jnp.int32))
        return gt, eq

    zeros = jnp.zeros((16,), jnp.int32)
    gt0, eq0 = lax.fori_loop(0, shard * 64, pbody, (zeros, zeros))

    off = gt0 + jnp.minimum(eq0, rvec)
    eqs = eq0
    for j in range(64):
        kv = keys_v[pl.ds(start + j * 16, 16)]
        gtm = kv > tvec
        eqm = kv == tvec
        eqi = eqm.astype(jnp.int32)
        eq_excl = plsc.cumsum(eqi) - eqi
        sel = gtm | (eqm & ((eqs + eq_excl) < rvec))
        seli = sel.astype(jnp.int32)
        rank = plsc.cumsum(seli) - seli
        dump = bloc * STAGE_ROW + K + ((s * 16 + lane) % 128)
        pos = jnp.where(sel, bloc * STAGE_ROW + off + rank, dump)
        posb[j // 8, pl.ds((j % 8) * 16, 16)] = pos
        valb[j // 8, pl.ds((j % 8) * 16, 16)] = start + j * 16 + lane
        off = off + jnp.sum(seli)
        eqs = eqs + jnp.sum(eqi)

    for t in range(8):
        pltpu.sync_copy(valb.at[t], stage_sh.at[posb.at[t]])

    plsc.subcore_barrier()

    kbase = shard * 512
    pltpu.sync_copy(stage_sh.at[pl.ds(bloc * STAGE_ROW + kbase, 512)], idxmy)
    pltpu.sync_copy(idxmy, idx_ref.at[b, pl.ds(kbase, 512)])

    bufs = (rows_a, rows_b)
    desc = pltpu.async_copy(
        x_ref.at[b].at[idxmy.at[pl.ds(0, 64)]], bufs[0], gsem)
    for t in range(8):
        desc.wait()
        cur = bufs[t % 2]
        if t < 7:
            desc = pltpu.async_copy(
                x_ref.at[b].at[idxmy.at[pl.ds((t + 1) * 64, 64)]],
                bufs[(t + 1) % 2], gsem)
        pltpu.sync_copy(cur, xs_ref.at[b, pl.ds(kbase + t * 64, 64), :])


def _sc_stage(x, keys, tval, rval):
    mesh = plsc.VectorSubcoreMesh(core_axis_name="c", subcore_axis_name="s")
    out_type = (
        jax.ShapeDtypeStruct((B, K, C), jnp.float32),
        jax.ShapeDtypeStruct((B, K), jnp.int32),
    )
    xs, idx = pl.kernel(
        _sc_body,
        out_type,
        mesh=mesh,
        compiler_params=pltpu.CompilerParams(needs_layout_passes=False),
        scratch_types=[
            pltpu.VMEM((L,), jnp.float32),
            pltpu.VMEM((128,), jnp.float32),
            pltpu.VMEM((128,), jnp.int32),
            pltpu.VMEM((8, 128), jnp.int32),
            pltpu.VMEM((8, 128), jnp.int32),
            pltpu.VMEM((512,), jnp.int32),
            pltpu.VMEM((64, C), jnp.float32),
            pltpu.VMEM((64, C), jnp.float32),
            pltpu.VMEM_SHARED((2 * STAGE_ROW,), jnp.int32),
            pltpu.SemaphoreType.DMA,
        ],
    )(x, keys, tval, rval)
    return xs, idx


def kernel(x):
    keys, tval, rval = _tc_stage(x)
    xs, idx = _sc_stage(x, keys, tval, rval)
    return (xs, idx)
